# baseline (speedup 1.0000x reference)
import functools

import jax
import jax.numpy as jnp
from jax import lax
from jax.experimental import pallas as pl
from jax.experimental.pallas import tpu as pltpu
from jax.experimental.pallas import tpu_sc as plsc

BATCH = 16
PROMPT_LEN = 128
HIDDEN = 2048
NUM_CODES = 8192
CODE_DIM = 64

TOKENS = BATCH * PROMPT_LEN
TILE = 512
TILE_DEC = 256


def _encode_body(x_ref, w_enc_ref, b_enc_ref, cb_ref, z_e_ref, idx_ref,
                 idx_pair_ref):
    x = x_ref[...]
    cb = cb_ref[...]

    z_e = (
        lax.dot_general(x, w_enc_ref[...], (((1,), (0,)), ((), ())),
                        preferred_element_type=jnp.float32)
        + b_enc_ref[...]
    )
    z_e_ref[...] = z_e

    zz = jnp.sum(z_e * z_e, axis=1, keepdims=True)
    cross = lax.dot_general(z_e, cb, (((1,), (1,)), ((), ())),
                            preferred_element_type=jnp.float32)
    cc = jnp.sum(cb * cb, axis=1)
    d2 = (zz - 2.0 * cross) + cc[None, :]
    idx = jnp.argmin(d2, axis=1).astype(jnp.int32)
    idx_ref[...] = idx[None, None, :]
    idx_pair_ref[...] = lax.shift_right_logical(idx, 1)[None, None, :]


def _decode_body(z_e_ref, z_q2_ref, idx_ref, w_dec_ref, b_dec_ref, out_ref):
    z_e = z_e_ref[...]
    z_q2 = z_q2_ref[...]
    parity = (idx_ref[0, 0, :] & 1)[:, None]
    z_q = jnp.where(parity == 1,
                    z_q2[:, CODE_DIM:],
                    z_q2[:, :CODE_DIM])
    z_q_st = z_e + (z_q - z_e)
    out_ref[...] = (
        lax.dot_general(z_q_st, w_dec_ref[...], (((1,), (0,)), ((), ())),
                        preferred_element_type=jnp.float32)
        + b_dec_ref[...]
    )


def _sc_gather(table, idx, width):
    info = plsc.get_sparse_core_info()
    nc, ns = info.num_cores, info.num_subcores
    rows_per_worker = TOKENS // (nc * ns)
    mesh = plsc.VectorSubcoreMesh(core_axis_name="c", subcore_axis_name="s")

    @functools.partial(
        pl.kernel,
        mesh=mesh,
        out_type=jax.ShapeDtypeStruct((TOKENS, width), jnp.float32),
        scratch_types=[
            pltpu.VMEM((rows_per_worker,), jnp.int32),
            pltpu.VMEM((rows_per_worker, width), jnp.float32),
            pltpu.SemaphoreType.DMA,
        ],
    )
    def gather(table_hbm, idx_hbm, out_hbm, idx_v, rows_v, sem):
        wid = lax.axis_index("s") * nc + lax.axis_index("c")
        base = wid * rows_per_worker
        pltpu.sync_copy(idx_hbm.at[pl.ds(base, rows_per_worker)], idx_v)
        pltpu.async_copy(table_hbm.at[idx_v], rows_v, sem).wait()
        pltpu.sync_copy(rows_v, out_hbm.at[pl.ds(base, rows_per_worker)])

    return gather(table, idx)


@functools.partial(jax.jit, static_argnames=("interpret",))
def kernel(task_des, W_enc, b_enc, codebook, W_dec, b_dec, interpret=False):
    x = task_des.reshape(TOKENS, HIDDEN)
    n_enc = TOKENS // TILE
    z_e, idx3, idx_pair3 = pl.pallas_call(
        _encode_body,
        grid=(n_enc,),
        in_specs=[
            pl.BlockSpec((TILE, HIDDEN), lambda i: (i, 0)),
            pl.BlockSpec((HIDDEN, CODE_DIM), lambda i: (0, 0)),
            pl.BlockSpec((1, CODE_DIM), lambda i: (0, 0)),
            pl.BlockSpec((NUM_CODES, CODE_DIM), lambda i: (0, 0)),
        ],
        out_specs=[
            pl.BlockSpec((TILE, CODE_DIM), lambda i: (i, 0)),
            pl.BlockSpec((1, 1, TILE), lambda i: (i, 0, 0)),
            pl.BlockSpec((1, 1, TILE), lambda i: (i, 0, 0)),
        ],
        out_shape=[
            jax.ShapeDtypeStruct((TOKENS, CODE_DIM), jnp.float32),
            jax.ShapeDtypeStruct((n_enc, 1, TILE), jnp.int32),
            jax.ShapeDtypeStruct((n_enc, 1, TILE), jnp.int32),
        ],
        compiler_params=pltpu.CompilerParams(
            dimension_semantics=("parallel",)),
        interpret=interpret,
    )(x, W_enc, b_enc.reshape(1, CODE_DIM), codebook)

    table2 = codebook.reshape(NUM_CODES // 2, 2 * CODE_DIM)
    idx_pair = idx_pair3.reshape(TOKENS)
    if interpret:
        z_q2 = jnp.take(table2, idx_pair, axis=0)
    else:
        z_q2 = _sc_gather(table2, idx_pair, 2 * CODE_DIM)

    n_dec = TOKENS // TILE_DEC
    idx3d = idx3.reshape(n_dec, 1, TILE_DEC)
    out = pl.pallas_call(
        _decode_body,
        grid=(n_dec,),
        in_specs=[
            pl.BlockSpec((TILE_DEC, CODE_DIM), lambda i: (i, 0)),
            pl.BlockSpec((TILE_DEC, 2 * CODE_DIM), lambda i: (i, 0)),
            pl.BlockSpec((1, 1, TILE_DEC), lambda i: (i, 0, 0)),
            pl.BlockSpec((CODE_DIM, HIDDEN), lambda i: (0, 0)),
            pl.BlockSpec((1, HIDDEN), lambda i: (0, 0)),
        ],
        out_specs=pl.BlockSpec((TILE_DEC, HIDDEN), lambda i: (i, 0)),
        out_shape=jax.ShapeDtypeStruct((TOKENS, HIDDEN), jnp.float32),
        compiler_params=pltpu.CompilerParams(
            dimension_semantics=("parallel",)),
        interpret=interpret,
    )(z_e, z_q2, idx3d, W_dec, b_dec.reshape(1, HIDDEN))
    return out.reshape(BATCH, PROMPT_LEN, HIDDEN)


# speedup vs baseline: 1.4241x; 1.4241x over previous
import functools

import jax
import jax.numpy as jnp
from jax import lax
from jax.experimental import pallas as pl
from jax.experimental.pallas import tpu as pltpu

BATCH = 16
PROMPT_LEN = 128
HIDDEN = 2048
NUM_CODES = 8192
CODE_DIM = 64

TOKENS = BATCH * PROMPT_LEN
TILE = 512
K_CHUNK = 1024


def _body(x_ref, w_enc_ref, b_enc_ref, cb_ref, t4_ref, w_dec_ref, b_dec_ref,
          out_ref):
    x = x_ref[...]
    w_enc = w_enc_ref[...]
    cb = cb_ref[...]

    z_e = (
        lax.dot_general(x, w_enc, (((1,), (0,)), ((), ())),
                        preferred_element_type=jnp.float32)
        + b_enc_ref[...]
    )

    zz = jnp.sum(z_e * z_e, axis=1, keepdims=True)
    cross = lax.dot_general(z_e, cb, (((1,), (1,)), ((), ())),
                            preferred_element_type=jnp.float32)
    cc = jnp.sum(cb * cb, axis=1)
    d2 = (zz - 2.0 * cross) + cc[None, :]
    idx = jnp.argmin(d2, axis=1).astype(jnp.int32)

    idx_q = lax.shift_right_logical(idx, 2)
    q_iota = lax.broadcasted_iota(jnp.int32, (TILE, NUM_CODES // 4), 1)
    onehot4 = (q_iota == idx_q[:, None]).astype(jnp.float32)
    z_q4 = lax.dot_general(onehot4, t4_ref[...], (((1,), (0,)), ((), ())),
                           preferred_element_type=jnp.float32)
    b0 = (idx & 1)[:, None] == 1
    b1 = (idx & 2)[:, None] == 2
    s0 = z_q4[:, 0 * CODE_DIM:1 * CODE_DIM]
    s1 = z_q4[:, 1 * CODE_DIM:2 * CODE_DIM]
    s2 = z_q4[:, 2 * CODE_DIM:3 * CODE_DIM]
    s3 = z_q4[:, 3 * CODE_DIM:4 * CODE_DIM]
    z_q = jnp.where(b1, jnp.where(b0, s3, s2), jnp.where(b0, s1, s0))

    z_q_st = z_e + (z_q - z_e)

    out_ref[...] = (
        lax.dot_general(z_q_st, w_dec_ref[...], (((1,), (0,)), ((), ())),
                        preferred_element_type=jnp.float32)
        + b_dec_ref[...]
    )


@functools.partial(jax.jit, static_argnames=("interpret",))
def kernel(task_des, W_enc, b_enc, codebook, W_dec, b_dec, interpret=False):
    x = task_des.reshape(TOKENS, HIDDEN)
    grid = (TOKENS // TILE,)
    out = pl.pallas_call(
        _body,
        grid=grid,
        in_specs=[
            pl.BlockSpec((TILE, HIDDEN), lambda i: (i, 0)),
            pl.BlockSpec((HIDDEN, CODE_DIM), lambda i: (0, 0)),
            pl.BlockSpec((1, CODE_DIM), lambda i: (0, 0)),
            pl.BlockSpec((NUM_CODES, CODE_DIM), lambda i: (0, 0)),
            pl.BlockSpec((NUM_CODES // 4, 4 * CODE_DIM), lambda i: (0, 0)),
            pl.BlockSpec((CODE_DIM, HIDDEN), lambda i: (0, 0)),
            pl.BlockSpec((1, HIDDEN), lambda i: (0, 0)),
        ],
        out_specs=pl.BlockSpec((TILE, HIDDEN), lambda i: (i, 0)),
        out_shape=jax.ShapeDtypeStruct((TOKENS, HIDDEN), jnp.float32),
        compiler_params=pltpu.CompilerParams(
            dimension_semantics=("parallel",)),
        interpret=interpret,
    )(x, W_enc, b_enc.reshape(1, CODE_DIM), codebook,
      codebook.reshape(NUM_CODES // 4, 4 * CODE_DIM), W_dec,
      b_dec.reshape(1, HIDDEN))
    return out.reshape(BATCH, PROMPT_LEN, HIDDEN)
